# baseline (speedup 1.0000x reference)
import functools

import jax
import jax.numpy as jnp
from jax import lax
from jax.experimental import pallas as pl
from jax.experimental.pallas import tpu as pltpu
from jax.experimental.pallas import tpu_sc as plsc

SEQ = 8192
EMB = 64
FIRE = 16

_cached = None


def _build():
    global _cached
    if _cached is not None:
        return _cached

    info = plsc.get_sparse_core_info()
    nc, ns = info.num_cores, info.num_subcores
    nw = nc * ns
    bpw = SEQ // nw
    mesh = plsc.VectorSubcoreMesh(core_axis_name="c", subcore_axis_name="s")

    @functools.partial(
        pl.kernel,
        mesh=mesh,
        out_type=jax.ShapeDtypeStruct((SEQ, EMB), jnp.float32),
        scratch_types=[
            pltpu.VMEM((bpw,), jnp.int32),
            pltpu.VMEM((bpw, EMB), jnp.float32),
            pltpu.VMEM((bpw, EMB), jnp.float32),
            pltpu.SemaphoreType.DMA,
            pltpu.SemaphoreType.DMA,
        ],
        compiler_params=pltpu.CompilerParams(skip_device_barrier=True),
    )
    def k(x_hbm, tok_hbm, pos_hbm, out_hbm, idx_v, rows_v, pos_v, sem, gsem):
        wid = lax.axis_index("s") * nc + lax.axis_index("c")
        base = wid * bpw
        pltpu.sync_copy(x_hbm.at[pl.ds(base, bpw)], idx_v)
        pcp = pltpu.async_copy(
            pos_hbm.at[pl.ds(SEQ - base - bpw, bpw)], pos_v, sem
        )

        def fetch(g, carry):
            jj = g * FIRE
            vec = idx_v[pl.ds(jj, FIRE)]
            cps = []
            for b in range(FIRE):
                r = vec[b]
                cps.append(
                    pltpu.async_copy(
                        tok_hbm.at[pl.ds(r, 1)],
                        rows_v.at[pl.ds(jj + b, 1)],
                        gsem,
                    )
                )
            for cp in cps:
                cp.wait()
            return carry

        lax.fori_loop(0, bpw // FIRE, fetch, 0)
        pcp.wait()

        def body(j, carry):
            rj = bpw - 1 - j
            for c in range(EMB // 16):
                sl = pl.ds(c * 16, 16)
                rows_v[j, sl] = rows_v[j, sl] + pos_v[rj, sl]
            return carry

        lax.fori_loop(0, bpw, body, 0)
        pltpu.sync_copy(rows_v, out_hbm.at[pl.ds(base, bpw)])

    _cached = k
    return _cached


def kernel(x, token_table, pos_table):
    return _build()(x.astype(jnp.int32), token_table, pos_table)


# speedup vs baseline: 2.8259x; 2.8259x over previous
import functools

import jax
import jax.numpy as jnp
from jax import lax
from jax.experimental import pallas as pl
from jax.experimental.pallas import tpu as pltpu
from jax.experimental.pallas import tpu_sc as plsc

SEQ = 8192
EMB = 64
VOCAB = 1000000
NBUF = 8

_cached = None


def _build():
    global _cached
    if _cached is not None:
        return _cached

    info = plsc.get_sparse_core_info()
    nc, ns = info.num_cores, info.num_subcores
    nw = nc * ns
    bpw = SEQ // nw
    mesh = plsc.VectorSubcoreMesh(core_axis_name="c", subcore_axis_name="s")

    @functools.partial(
        pl.kernel,
        mesh=mesh,
        out_type=jax.ShapeDtypeStruct((EMB, SEQ), jnp.float32),
        scratch_types=[
            pltpu.VMEM((bpw,), jnp.int32),
            pltpu.VMEM((EMB, bpw), jnp.float32),
            pltpu.VMEM((EMB, bpw), jnp.float32),
            [pltpu.VMEM((EMB, 128), jnp.float32) for _ in range(NBUF)],
            pltpu.SemaphoreType.DMA,
            pltpu.SemaphoreType.DMA,
        ],
        compiler_params=pltpu.CompilerParams(needs_layout_passes=False),
    )
    def k(x_hbm, tokt_hbm, post_hbm, outt_hbm,
          idx_v, pos_v, out_v, bufs, sem, gsem):
        wid = lax.axis_index("s") * nc + lax.axis_index("c")
        base = wid * bpw
        iota16 = lax.iota(jnp.int32, 16)
        pltpu.sync_copy(x_hbm.at[pl.ds(base, bpw)], idx_v)
        pcp = pltpu.async_copy(
            post_hbm.at[:, pl.ds(SEQ - base - bpw, bpw)], pos_v, sem
        )

        def fetch_group(gg, carry):
            vec = idx_v[pl.ds(gg * 16, 16)]
            for sub in range(16 // NBUF):
                cps = []
                for b in range(NBUF):
                    col = vec[sub * NBUF + b]
                    c0 = pl.multiple_of((col >> 7) * 128, 128)
                    cps.append(
                        pltpu.async_copy(
                            tokt_hbm.at[:, pl.ds(c0, 128)], bufs[b], gsem
                        )
                    )
                for b in range(NBUF):
                    col = vec[sub * NBUF + b]
                    lane = jnp.broadcast_to(col & 127, (16,))
                    j = jnp.broadcast_to(gg * 16 + sub * NBUF + b, (16,))
                    cps[b].wait()
                    for p in range(EMB // 16):
                        rows = iota16 + p * 16
                        vals = plsc.load_gather(bufs[b], [rows, lane])
                        plsc.store_scatter(out_v, [rows, j], vals)
            return carry

        lax.fori_loop(0, bpw // 16, fetch_group, 0)
        pcp.wait()

        def body(d, carry):
            for g in range(bpw // 16):
                sl = pl.ds(g * 16, 16)
                rsl = pl.ds(bpw - (g + 1) * 16, 16)
                out_v[d, sl] = out_v[d, sl] + lax.rev(pos_v[d, rsl], (0,))
            return carry

        lax.fori_loop(0, EMB, body, 0)
        pltpu.sync_copy(out_v, outt_hbm.at[:, pl.ds(base, bpw)])

    _cached = k
    return _cached


def kernel(x, token_table, pos_table):
    outt = _build()(x.astype(jnp.int32), token_table.T, pos_table.T)
    return outt.T


# speedup vs baseline: 3.0398x; 1.0757x over previous
import functools

import jax
import jax.numpy as jnp
from jax import lax
from jax.experimental import pallas as pl
from jax.experimental.pallas import tpu as pltpu
from jax.experimental.pallas import tpu_sc as plsc

SEQ = 8192
EMB = 64
VOCAB = 1000000
NBUF = 8

_cached = None


def _build():
    global _cached
    if _cached is not None:
        return _cached

    info = plsc.get_sparse_core_info()
    nc, ns = info.num_cores, info.num_subcores
    nw = nc * ns
    bpw = SEQ // nw
    mesh = plsc.VectorSubcoreMesh(core_axis_name="c", subcore_axis_name="s")

    @functools.partial(
        pl.kernel,
        mesh=mesh,
        out_type=jax.ShapeDtypeStruct((EMB, SEQ), jnp.float32),
        scratch_types=[
            pltpu.VMEM((bpw,), jnp.int32),
            pltpu.VMEM((EMB, bpw), jnp.float32),
            pltpu.VMEM((EMB, bpw), jnp.float32),
            [pltpu.VMEM((EMB, 128), jnp.float32) for _ in range(NBUF)],
            pltpu.SemaphoreType.DMA,
            pltpu.SemaphoreType.DMA,
        ],
        compiler_params=pltpu.CompilerParams(needs_layout_passes=False),
    )
    def k(x_hbm, tokt_hbm, post_hbm, outt_hbm,
          idx_v, pos_v, out_v, bufs, sem, gsem):
        wid = lax.axis_index("s") * nc + lax.axis_index("c")
        base = wid * bpw
        iota16 = lax.iota(jnp.int32, 16)
        pltpu.sync_copy(x_hbm.at[pl.ds(base, bpw)], idx_v)
        pcp = pltpu.async_copy(
            post_hbm.at[:, pl.ds(SEQ - base - bpw, bpw)], pos_v, sem
        )

        def fire(vec, t, b):
            col = vec[t]
            c0 = pl.multiple_of((col >> 7) * 128, 128)
            return pltpu.async_copy(
                tokt_hbm.at[:, pl.ds(c0, 128)], bufs[b], gsem
            )

        def drain_extract(cp, vec, t, j0, b):
            col = vec[t]
            lane = jnp.broadcast_to(col & 127, (16,))
            j = jnp.broadcast_to(j0 + t, (16,))
            cp.wait()
            for p in range(EMB // 16):
                rows = iota16 + p * 16
                vals = plsc.load_gather(bufs[b], [rows, lane])
                plsc.store_scatter(out_v, [rows, j], vals)

        def fetch_group(gg, carry):
            vec = idx_v[pl.ds(gg * 16, 16)]
            j0 = gg * 16
            cps = [fire(vec, b, b) for b in range(NBUF)]
            for b in range(16 - NBUF):
                drain_extract(cps[b], vec, b, j0, b % NBUF)
                cps.append(fire(vec, NBUF + b, b % NBUF))
            for b in range(16 - NBUF, 16):
                drain_extract(cps[b], vec, b, j0, b % NBUF)
            return carry

        lax.fori_loop(0, bpw // 16, fetch_group, 0)
        pcp.wait()

        def body(d, carry):
            for g in range(bpw // 16):
                sl = pl.ds(g * 16, 16)
                rsl = pl.ds(bpw - (g + 1) * 16, 16)
                out_v[d, sl] = out_v[d, sl] + lax.rev(pos_v[d, rsl], (0,))
            return carry

        lax.fori_loop(0, EMB, body, 0)
        pltpu.sync_copy(out_v, outt_hbm.at[:, pl.ds(base, bpw)])

    _cached = k
    return _cached


def kernel(x, token_table, pos_table):
    outt = _build()(x.astype(jnp.int32), token_table.T, pos_table.T)
    return outt.T
